# baseline (speedup 1.0000x reference)
import jax
import jax.numpy as jnp
from jax import lax
from jax.experimental import pallas as pl
from jax.experimental.pallas import tpu as pltpu
from jax.experimental.pallas import tpu_sc as plsc

N_NODES = 10000
N_EDGES = 320000
D = 128
DH = D // 2

NC = 2
NS = 16
NW = NC * NS
EDGES_PER_TILE = N_EDGES // NW
CHUNK = 80
NCHUNK = EDGES_PER_TILE // CHUNK
NBUF = 5
ACC_ROWS = 10240
ROWS_PER_TILE = ACC_ROWS // NS


def _sc_agg_body(h_hbm, src_hbm, dst_hbm, zeros_hbm, out_hbm,
                 src_v, dst_v, *rest):
    bufs = list(rest[:NBUF])
    acc = rest[NBUF]
    gsems = list(rest[NBUF + 1:2 * NBUF + 1])
    ssems = list(rest[2 * NBUF + 1:3 * NBUF + 1])

    c = lax.axis_index("c")
    s = lax.axis_index("s")
    wid = s * NC + c
    rows = pl.ds(s * ROWS_PER_TILE, ROWS_PER_TILE)

    pltpu.sync_copy(src_hbm.at[wid], src_v)
    pltpu.sync_copy(dst_hbm.at[wid], dst_v)

    for p in range(2):
        table = h_hbm.at[p]
        pltpu.sync_copy(zeros_hbm, acc.at[rows])
        for b in range(NBUF):
            pltpu.async_copy(table.at[src_v.at[b]], bufs[b], gsems[b])
        plsc.subcore_barrier()

        @pl.loop(0, NCHUNK, step=NBUF)
        def _chunks(j0):
            for b in range(NBUF):
                j = j0 + b
                pb = (b - 1) % NBUF
                pltpu.make_async_copy(table.at[src_v.at[j]], bufs[b], gsems[b]).wait()
                pltpu.async_copy(bufs[b], acc.at[dst_v.at[j]], ssems[b], add=True)

                @pl.when((j >= 1) & (j + NBUF - 1 < NCHUNK))
                def _issue(b=b, pb=pb, j=j):
                    pltpu.make_async_copy(
                        bufs[pb], acc.at[dst_v.at[j - 1]], ssems[pb]).wait()
                    pltpu.async_copy(
                        table.at[src_v.at[j + NBUF - 1]], bufs[pb], gsems[pb])

        for b in range(NBUF):
            j = NCHUNK - NBUF + b
            pltpu.make_async_copy(bufs[b], acc.at[dst_v.at[j]], ssems[b]).wait()

        plsc.subcore_barrier()

        pltpu.sync_copy(acc.at[rows], out_hbm.at[p, c, rows])


_sc_agg = pl.kernel(
    _sc_agg_body,
    out_type=jax.ShapeDtypeStruct((2, NC, ACC_ROWS, DH), jnp.float32),
    mesh=plsc.VectorSubcoreMesh(core_axis_name="c", subcore_axis_name="s"),
    compiler_params=pltpu.CompilerParams(use_tc_tiling_on_sc=False),
    scratch_types=(
        [pltpu.VMEM((NCHUNK, CHUNK), jnp.int32)] * 2
        + [pltpu.VMEM((CHUNK, DH), jnp.float32)] * NBUF
        + [pltpu.VMEM_SHARED((ACC_ROWS, DH), jnp.float32)]
        + [pltpu.SemaphoreType.DMA] * (2 * NBUF)
    ),
)


def _mlp_body(x_ref, paa_ref, pab_ref, pba_ref, pbb_ref,
              w1_ref, b1_ref, w2_ref, b2_ref, o_ref, opair_ref):
    agg_lo = paa_ref[0, 0] + pab_ref[0, 0]
    agg_hi = pba_ref[0, 0] + pbb_ref[0, 0]
    h = x_ref[...] + jnp.concatenate([agg_lo, agg_hi], axis=1)
    h = jnp.dot(h, w1_ref[...], preferred_element_type=jnp.float32)
    h = jnp.maximum(h + b1_ref[0:1, :], 0.0)
    h = jnp.dot(h, w2_ref[...], preferred_element_type=jnp.float32)
    h = jnp.maximum(h + b2_ref[0:1, :], 0.0)
    o_ref[...] = h
    opair_ref[0, :, :] = h[:, :DH]
    opair_ref[1, :, :] = h[:, DH:]


_MLP_BLOCK = 1000

_mlp = pl.pallas_call(
    _mlp_body,
    grid=(N_NODES // _MLP_BLOCK,),
    in_specs=[
        pl.BlockSpec((_MLP_BLOCK, D), lambda i: (i, 0)),
        pl.BlockSpec((1, 1, _MLP_BLOCK, DH), lambda i: (0, 0, i, 0)),
        pl.BlockSpec((1, 1, _MLP_BLOCK, DH), lambda i: (0, 1, i, 0)),
        pl.BlockSpec((1, 1, _MLP_BLOCK, DH), lambda i: (1, 0, i, 0)),
        pl.BlockSpec((1, 1, _MLP_BLOCK, DH), lambda i: (1, 1, i, 0)),
        pl.BlockSpec((D, D), lambda i: (0, 0)),
        pl.BlockSpec((8, D), lambda i: (0, 0)),
        pl.BlockSpec((D, D), lambda i: (0, 0)),
        pl.BlockSpec((8, D), lambda i: (0, 0)),
    ],
    out_specs=[
        pl.BlockSpec((_MLP_BLOCK, D), lambda i: (i, 0)),
        pl.BlockSpec((2, _MLP_BLOCK, DH), lambda i: (0, i, 0)),
    ],
    out_shape=[
        jax.ShapeDtypeStruct((N_NODES, D), jnp.float32),
        jax.ShapeDtypeStruct((2, N_NODES, DH), jnp.float32),
    ],
)


def kernel(x, edge_index, W1_0, b1_0, W2_0, b2_0, W1_1, b1_1, W2_1, b2_1,
           W1_2, b1_2, W2_2, b2_2):
    src = edge_index[0].astype(jnp.int32).reshape(NW, NCHUNK, CHUNK)
    dst = edge_index[1].astype(jnp.int32).reshape(NW, NCHUNK, CHUNK)
    zeros = jnp.zeros((ROWS_PER_TILE, DH), jnp.float32)

    Ws1 = jnp.stack([W1_0, W1_1, W1_2])
    Ws2 = jnp.stack([W2_0, W2_1, W2_2])
    bs1 = jnp.stack([jnp.broadcast_to(b.reshape(1, D), (8, D))
                     for b in (b1_0, b1_1, b1_2)])
    bs2 = jnp.stack([jnp.broadcast_to(b.reshape(1, D), (8, D))
                     for b in (b2_0, b2_1, b2_2)])

    def step(carry, p):
        h, h_pair = carry
        W1, b1, W2, b2 = p
        part = _sc_agg(h_pair, src, dst, zeros)
        hn, hn_pair = _mlp(h, part, part, part, part, W1, b1, W2, b2)
        return (hn, hn_pair), hn

    x_pair = jnp.stack([x[:, :DH], x[:, DH:]])
    _, ys = lax.scan(step, (x, x_pair), (Ws1, bs1, Ws2, bs2))
    return jnp.concatenate([x, ys[0], ys[1], ys[2]], axis=1)


# speedup vs baseline: 1.0371x; 1.0371x over previous
import jax
import jax.numpy as jnp
from jax import lax
from jax.experimental import pallas as pl
from jax.experimental.pallas import tpu as pltpu
from jax.experimental.pallas import tpu_sc as plsc

N_NODES = 10000
N_EDGES = 320000
D = 128
DH = D // 2

NC = 2
NS = 16
NW = NC * NS
EDGES_PER_TILE = N_EDGES // NW
CHUNK = 80
NCHUNK = EDGES_PER_TILE // CHUNK
NBUF = 5
ACC_ROWS = 10240
ROWS_PER_TILE = ACC_ROWS // NS


def _sc_agg_body(h_hbm, src_hbm, dst_hbm, zeros_hbm, out_hbm,
                 src_v, dst_v, *rest):
    bufs = list(rest[:NBUF])
    acc = rest[NBUF]
    gsems = list(rest[NBUF + 1:2 * NBUF + 1])
    ssems = list(rest[2 * NBUF + 1:3 * NBUF + 1])

    c = lax.axis_index("c")
    s = lax.axis_index("s")
    wid = s * NC + c
    rows = pl.ds(s * ROWS_PER_TILE, ROWS_PER_TILE)

    pltpu.sync_copy(src_hbm.at[wid], src_v)
    pltpu.sync_copy(dst_hbm.at[wid], dst_v)

    for p in range(2):
        table = h_hbm.at[p]
        pltpu.sync_copy(zeros_hbm, acc.at[rows])
        for b in range(NBUF):
            pltpu.async_copy(table.at[src_v.at[b]], bufs[b], gsems[b])
        plsc.subcore_barrier()

        @pl.loop(0, NCHUNK, step=NBUF)
        def _chunks(j0):
            for b in range(NBUF):
                j = j0 + b
                pltpu.make_async_copy(table.at[src_v.at[j]], bufs[b], gsems[b]).wait()
                pltpu.async_copy(bufs[b], acc.at[dst_v.at[j]], ssems[b], add=True)

                @pl.when(j + NBUF < NCHUNK)
                def _issue(b=b, j=j):
                    pltpu.make_async_copy(bufs[b], acc.at[dst_v.at[j]], ssems[b]).wait()
                    pltpu.async_copy(table.at[src_v.at[j + NBUF]], bufs[b], gsems[b])

        for b in range(NBUF):
            j = NCHUNK - NBUF + b
            pltpu.make_async_copy(bufs[b], acc.at[dst_v.at[j]], ssems[b]).wait()

        plsc.subcore_barrier()

        pltpu.sync_copy(acc.at[rows], out_hbm.at[p, c, rows])


_sc_agg = pl.kernel(
    _sc_agg_body,
    out_type=jax.ShapeDtypeStruct((2, NC, ACC_ROWS, DH), jnp.float32),
    mesh=plsc.VectorSubcoreMesh(core_axis_name="c", subcore_axis_name="s"),
    compiler_params=pltpu.CompilerParams(use_tc_tiling_on_sc=False),
    scratch_types=(
        [pltpu.VMEM((NCHUNK, CHUNK), jnp.int32)] * 2
        + [pltpu.VMEM((CHUNK, DH), jnp.float32)] * NBUF
        + [pltpu.VMEM_SHARED((ACC_ROWS, DH), jnp.float32)]
        + [pltpu.SemaphoreType.DMA] * (2 * NBUF)
    ),
)


def _mlp_body(x_ref, paa_ref, pab_ref, pba_ref, pbb_ref,
              w1_ref, b1_ref, w2_ref, b2_ref, o_ref, opair_ref):
    agg_lo = paa_ref[0, 0] + pab_ref[0, 0]
    agg_hi = pba_ref[0, 0] + pbb_ref[0, 0]
    h = x_ref[...] + jnp.concatenate([agg_lo, agg_hi], axis=1)
    h = jnp.dot(h, w1_ref[...], preferred_element_type=jnp.float32)
    h = jnp.maximum(h + b1_ref[0:1, :], 0.0)
    h = jnp.dot(h, w2_ref[...], preferred_element_type=jnp.float32)
    h = jnp.maximum(h + b2_ref[0:1, :], 0.0)
    o_ref[...] = h
    opair_ref[0, :, :] = h[:, :DH]
    opair_ref[1, :, :] = h[:, DH:]


_MLP_BLOCK = 2000

_mlp = pl.pallas_call(
    _mlp_body,
    grid=(N_NODES // _MLP_BLOCK,),
    in_specs=[
        pl.BlockSpec((_MLP_BLOCK, D), lambda i: (i, 0)),
        pl.BlockSpec((1, 1, _MLP_BLOCK, DH), lambda i: (0, 0, i, 0)),
        pl.BlockSpec((1, 1, _MLP_BLOCK, DH), lambda i: (0, 1, i, 0)),
        pl.BlockSpec((1, 1, _MLP_BLOCK, DH), lambda i: (1, 0, i, 0)),
        pl.BlockSpec((1, 1, _MLP_BLOCK, DH), lambda i: (1, 1, i, 0)),
        pl.BlockSpec((D, D), lambda i: (0, 0)),
        pl.BlockSpec((8, D), lambda i: (0, 0)),
        pl.BlockSpec((D, D), lambda i: (0, 0)),
        pl.BlockSpec((8, D), lambda i: (0, 0)),
    ],
    out_specs=[
        pl.BlockSpec((_MLP_BLOCK, D), lambda i: (i, 0)),
        pl.BlockSpec((2, _MLP_BLOCK, DH), lambda i: (0, i, 0)),
    ],
    out_shape=[
        jax.ShapeDtypeStruct((N_NODES, D), jnp.float32),
        jax.ShapeDtypeStruct((2, N_NODES, DH), jnp.float32),
    ],
)


def kernel(x, edge_index, W1_0, b1_0, W2_0, b2_0, W1_1, b1_1, W2_1, b2_1,
           W1_2, b1_2, W2_2, b2_2):
    src = edge_index[0].astype(jnp.int32).reshape(NW, NCHUNK, CHUNK)
    dst = edge_index[1].astype(jnp.int32).reshape(NW, NCHUNK, CHUNK)
    zeros = jnp.zeros((ROWS_PER_TILE, DH), jnp.float32)

    Ws1 = jnp.stack([W1_0, W1_1, W1_2])
    Ws2 = jnp.stack([W2_0, W2_1, W2_2])
    bs1 = jnp.stack([jnp.broadcast_to(b.reshape(1, D), (8, D))
                     for b in (b1_0, b1_1, b1_2)])
    bs2 = jnp.stack([jnp.broadcast_to(b.reshape(1, D), (8, D))
                     for b in (b2_0, b2_1, b2_2)])

    def step(carry, p):
        h, h_pair = carry
        W1, b1, W2, b2 = p
        part = _sc_agg(h_pair, src, dst, zeros)
        hn, hn_pair = _mlp(h, part, part, part, part, W1, b1, W2, b2)
        return (hn, hn_pair), hn

    x_pair = jnp.stack([x[:, :DH], x[:, DH:]])
    _, ys = lax.scan(step, (x, x_pair), (Ws1, bs1, Ws2, bs2))
    return jnp.concatenate([x, ys[0], ys[1], ys[2]], axis=1)
